# baseline (device time: 47927 ns/iter reference)
import jax
import jax.numpy as jnp
from jax import lax
from jax.experimental import pallas as pl
from jax.experimental.pallas import tpu as pltpu

N_DEV = 4
SUB = 4


def kernel(x, w_mat):
    m, _ = x.shape
    _, n = w_mat.shape
    m_per = m // N_DEV
    nh = n // 2
    subw = nh // SUB

    def body(
        x_hbm, w_hbm, out_hbm,
        x_vmem, w_vmem, part_cw, part_ccw, part_own,
        send_cw, recv_cw, send_ccw, recv_ccw, out_stage,
        in_sems, out_sems,
        send_sems_cw, recv_sems_cw, send_sems_ccw, recv_sems_ccw,
    ):
        my = lax.axis_index("i")
        right = lax.rem(my + 1, N_DEV)
        left = lax.rem(my + 3, N_DEV)

        blks = [left, right, lax.rem(my + 2, N_DEV), my]
        in_copies = []
        for j, c in enumerate(blks):
            cp = pltpu.make_async_copy(
                x_hbm.at[pl.ds(c * m_per, m_per), :],
                x_vmem.at[pl.ds(c * m_per, m_per), :],
                in_sems.at[j],
            )
            cp.start()
            in_copies.append(cp)
        for j, half in ((4, 0), (5, 1)):
            cp = pltpu.make_async_copy(
                w_hbm.at[:, pl.ds(half * nh, nh)],
                w_vmem.at[:, pl.ds(half * nh, nh)],
                in_sems.at[j],
            )
            cp.start()
            in_copies.append(cp)

        barrier_sem = pltpu.get_barrier_semaphore()
        for nbr in (left, right):
            pl.semaphore_signal(
                barrier_sem, inc=1,
                device_id=(nbr,), device_id_type=pl.DeviceIdType.MESH,
            )
        pl.semaphore_wait(barrier_sem, 2)

        def dot_half(c, half):
            xb = x_vmem[pl.ds(c * m_per, m_per), :].astype(jnp.bfloat16)
            wh = w_vmem[:, half * nh:(half + 1) * nh].astype(jnp.bfloat16)
            return lax.dot_general(
                xb, wh, (((1,), (0,)), ((), ())),
                preferred_element_type=jnp.float32,
            )

        def make_rdma(bufs, sems, s, k, dev):
            send_buf, recv_buf = bufs
            send_sems, recv_sems = sems
            return pltpu.make_async_remote_copy(
                src_ref=send_buf.at[s, k], dst_ref=recv_buf.at[s, k],
                send_sem=send_sems.at[s, k], recv_sem=recv_sems.at[s, k],
                device_id=(dev,), device_id_type=pl.DeviceIdType.MESH,
            )

        cw_bufs = (send_cw, recv_cw)
        cw_sems = (send_sems_cw, recv_sems_cw)
        ccw_bufs = (send_ccw, recv_ccw)
        ccw_sems = (send_sems_ccw, recv_sems_ccw)

        def c_cw(s):
            return lax.rem(my + (2 * N_DEV - 1 - s), N_DEV)

        def c_ccw(s):
            return lax.rem(my + 1 + s, N_DEV)

        rdmas = []

        def cols(k):
            return pl.ds(k * subw, subw)

        in_copies[0].wait()
        in_copies[4].wait()
        h0_cw = dot_half(c_cw(0), 0)
        for k in range(SUB):
            send_cw[0, k] = h0_cw[:, k * subw:(k + 1) * subw].astype(jnp.bfloat16)
            r = make_rdma(cw_bufs, cw_sems, 0, k, right)
            r.start()
            rdmas.append(r)
        in_copies[1].wait()
        in_copies[5].wait()
        h0_ccw = dot_half(c_ccw(0), 1)
        for k in range(SUB):
            send_ccw[0, k] = h0_ccw[:, k * subw:(k + 1) * subw].astype(jnp.bfloat16)
            r = make_rdma(ccw_bufs, ccw_sems, 0, k, left)
            r.start()
            rdmas.append(r)

        in_copies[2].wait()
        in_copies[3].wait()
        for s in (1, 2):
            part_cw[s - 1] = dot_half(c_cw(s), 0)
            part_ccw[s - 1] = dot_half(c_ccw(s), 1)
        part_own[:, 0:nh] = dot_half(my, 0)
        part_own[:, nh:n] = dot_half(my, 1)

        for s in (1, 2):
            for k in range(SUB):
                make_rdma(cw_bufs, cw_sems, s - 1, k, right).wait_recv()
                send_cw[s, k] = (
                    part_cw[s - 1, :, cols(k)]
                    + recv_cw[s - 1, k].astype(jnp.float32)
                ).astype(jnp.bfloat16)
                r = make_rdma(cw_bufs, cw_sems, s, k, right)
                r.start()
                rdmas.append(r)

                make_rdma(ccw_bufs, ccw_sems, s - 1, k, left).wait_recv()
                send_ccw[s, k] = (
                    part_ccw[s - 1, :, cols(k)]
                    + recv_ccw[s - 1, k].astype(jnp.float32)
                ).astype(jnp.bfloat16)
                r = make_rdma(ccw_bufs, ccw_sems, s, k, left)
                r.start()
                rdmas.append(r)

        cg = 0.7978845608028654

        def gelu(y):
            return 0.5 * y * (1.0 + jnp.tanh(cg * (y + 0.044715 * y * y * y)))

        out_copies = []
        for k in range(SUB):
            make_rdma(cw_bufs, cw_sems, 2, k, right).wait_recv()
            acc = part_own[:, cols(k)] + recv_cw[2, k].astype(jnp.float32)
            out_stage[0, k] = gelu(acc)
            cp = pltpu.make_async_copy(
                out_stage.at[0, k], out_hbm.at[:, cols(k)], out_sems.at[0, k],
            )
            cp.start()
            out_copies.append(cp)

            make_rdma(ccw_bufs, ccw_sems, 2, k, left).wait_recv()
            acc = (
                part_own[:, pl.ds(nh + k * subw, subw)]
                + recv_ccw[2, k].astype(jnp.float32)
            )
            out_stage[1, k] = gelu(acc)
            cp = pltpu.make_async_copy(
                out_stage.at[1, k],
                out_hbm.at[:, pl.ds(nh + k * subw, subw)],
                out_sems.at[1, k],
            )
            cp.start()
            out_copies.append(cp)

        for cp in out_copies:
            cp.wait()
        for r in rdmas:
            r.wait_send()

    return pl.pallas_call(
        body,
        out_shape=jax.ShapeDtypeStruct((m_per, n), jnp.float32),
        in_specs=[
            pl.BlockSpec(memory_space=pl.ANY),
            pl.BlockSpec(memory_space=pl.ANY),
        ],
        out_specs=pl.BlockSpec(memory_space=pl.ANY),
        scratch_shapes=[
            pltpu.VMEM((m, m // N_DEV), jnp.float32),
            pltpu.VMEM((m // N_DEV, n), jnp.float32),
            pltpu.VMEM((2, m_per, nh), jnp.float32),
            pltpu.VMEM((2, m_per, nh), jnp.float32),
            pltpu.VMEM((m_per, n), jnp.float32),
            pltpu.VMEM((3, SUB, m_per, subw), jnp.bfloat16),
            pltpu.VMEM((3, SUB, m_per, subw), jnp.bfloat16),
            pltpu.VMEM((3, SUB, m_per, subw), jnp.bfloat16),
            pltpu.VMEM((3, SUB, m_per, subw), jnp.bfloat16),
            pltpu.VMEM((2, SUB, m_per, subw), jnp.float32),
            pltpu.SemaphoreType.DMA((6,)),
            pltpu.SemaphoreType.DMA((2, SUB)),
            pltpu.SemaphoreType.DMA((3, SUB)),
            pltpu.SemaphoreType.DMA((3, SUB)),
            pltpu.SemaphoreType.DMA((3, SUB)),
            pltpu.SemaphoreType.DMA((3, SUB)),
        ],
        compiler_params=pltpu.CompilerParams(collective_id=0, vmem_limit_bytes=64 * 1024 * 1024),
    )(x, w_mat)
